# baseline (speedup 1.0000x reference)
import functools

import jax
import jax.numpy as jnp
from jax import lax
from jax.experimental import pallas as pl
from jax.experimental.pallas import tpu as pltpu

_N = 20000
_K = 6000
_NOUT = 300
_IOU = 0.7
_SCALE = 1000.0
_ROWS = 160
_LANES = 128
_P = _ROWS * _LANES
_C = 6144
_CROWS = 48
_NEG = -1e9


def _nms_body(c0_ref, c1_ref, c2_ref, c3_ref, s_ref, out_ref):
    f32 = jnp.float32
    i32 = jnp.int32
    imin = jnp.int32(-2147483648)

    row_i = lax.broadcasted_iota(i32, (_ROWS, _LANES), 0)
    lane_i = lax.broadcasted_iota(i32, (_ROWS, _LANES), 1)
    flat_i = row_i * _LANES + lane_i
    flat_c = lax.broadcasted_iota(i32, (_CROWS, _LANES), 0) * _LANES \
        + lax.broadcasted_iota(i32, (_CROWS, _LANES), 1)
    lane1 = lax.broadcasted_iota(i32, (1, _LANES), 1)
    valid = flat_i < _N

    b0 = c0_ref[:] * _SCALE
    b1 = c1_ref[:] * _SCALE
    b2 = c2_ref[:] * _SCALE
    b3 = c3_ref[:] * _SCALE
    x1 = jnp.minimum(b0, b2)
    x2 = jnp.maximum(b0, b2)
    y1 = jnp.minimum(b1, b3)
    y2 = jnp.maximum(b1, b3)
    scores = s_ref[:]

    bits = lax.bitcast_convert_type(scores, i32)
    akey = bits ^ (lax.shift_right_arithmetic(bits, 31) & jnp.int32(0x7FFFFFFF))
    akey = jnp.where(valid, akey, imin)
    inv = _P - flat_i

    Tf = imin
    Ti = jnp.int32(0)
    for b in range(31, -1, -1):
        trial = (Tf ^ imin) if b == 31 else (Tf | jnp.int32(1 << b))
        cnt = jnp.sum((akey >= trial).astype(i32))
        Tf = jnp.where(cnt >= _K, trial, Tf)
    for b in range(14, -1, -1):
        trial = Ti | jnp.int32(1 << b)
        cond = (akey > Tf) | ((akey == Tf) & (inv >= trial))
        cnt = jnp.sum(cond.astype(i32))
        Ti = jnp.where(cnt >= _K, trial, Ti)
    in_set = (akey > Tf) | ((akey == Tf) & (inv >= Ti))

    bf16 = jnp.bfloat16
    mask_bf = in_set.astype(bf16)
    up = (lax.broadcasted_iota(i32, (_LANES, _LANES), 0)
          < lax.broadcasted_iota(i32, (_LANES, _LANES), 1)).astype(bf16)
    lane_excl = lax.dot_general(mask_bf, up, (((1,), (0,)), ((), ())),
                                preferred_element_type=jnp.float32)
    rowsum = jnp.sum(in_set.astype(f32), axis=1, keepdims=True)
    lo = (lax.broadcasted_iota(i32, (_ROWS, _ROWS), 0)
          > lax.broadcasted_iota(i32, (_ROWS, _ROWS), 1)).astype(bf16)
    row_excl = lax.dot_general(lo, rowsum.astype(bf16), (((1,), (0,)), ((), ())),
                               preferred_element_type=jnp.float32)
    rank = (row_excl + lane_excl).astype(i32)

    d = jnp.where(in_set, flat_i - rank, 0)

    def shift_down(v, k):
        r, l = divmod(k, _LANES)
        if l:
            a = jnp.concatenate([v[:, l:], v[:, :l]], axis=1)
            b_ = jnp.concatenate([a[1:, :], a[:1, :]], axis=0)
            v = jnp.where(lane_i < _LANES - l, a, b_)
        if r:
            v = jnp.concatenate([v[r:, :], v[:r, :]], axis=0)
        return v

    for b in range(14):
        k = 1 << b
        dr = shift_down(d, k)
        mv = (lax.shift_right_logical(dr, b) & 1) == 1
        x1 = jnp.where(mv, shift_down(x1, k), x1)
        y1 = jnp.where(mv, shift_down(y1, k), y1)
        x2 = jnp.where(mv, shift_down(x2, k), x2)
        y2 = jnp.where(mv, shift_down(y2, k), y2)
        scores = jnp.where(mv, shift_down(scores, k), scores)
        d = jnp.where(mv, dr, d)

    x1c = x1[:_CROWS]
    y1c = y1[:_CROWS]
    x2c = x2[:_CROWS]
    y2c = y2[:_CROWS]
    scc = scores[:_CROWS]
    areas = (x2c - x1c) * (y2c - y1c)
    s0 = jnp.where(flat_c < _K, scc, f32(_NEG))
    neg_inf = f32(-jnp.inf)

    def lanered(t, op):
        for sh in (64, 32, 16, 8, 4, 2, 1):
            t = op(t, jnp.concatenate([t[:, sh:], t[:, :sh]], axis=1))
        return t

    def step(i, carry):
        s, dx1, dy1, dx2, dy2, ds = carry
        m = lanered(jnp.max(s, axis=0, keepdims=True), jnp.maximum)
        idx = lanered(jnp.min(jnp.where(s == m, flat_c, _C), axis=0,
                              keepdims=True), jnp.minimum)
        mask2 = flat_c == idx
        sx1 = lanered(jnp.max(jnp.where(mask2, x1c, neg_inf), axis=0,
                              keepdims=True), jnp.maximum)
        sy1 = lanered(jnp.max(jnp.where(mask2, y1c, neg_inf), axis=0,
                              keepdims=True), jnp.maximum)
        sx2 = lanered(jnp.max(jnp.where(mask2, x2c, neg_inf), axis=0,
                              keepdims=True), jnp.maximum)
        sy2 = lanered(jnp.max(jnp.where(mask2, y2c, neg_inf), axis=0,
                              keepdims=True), jnp.maximum)
        ssc = lanered(jnp.max(jnp.where(mask2, scc, neg_inf), axis=0,
                              keepdims=True), jnp.maximum)

        is_deg = m == f32(_NEG)
        dx1 = jnp.where(i == 0, sx1, dx1)
        dy1 = jnp.where(i == 0, sy1, dy1)
        dx2 = jnp.where(i == 0, sx2, dx2)
        dy2 = jnp.where(i == 0, sy2, dy2)
        ds = jnp.where(i == 0, ssc, ds)
        sx1 = jnp.where(is_deg, dx1, sx1)
        sy1 = jnp.where(is_deg, dy1, sy1)
        sx2 = jnp.where(is_deg, dx2, sx2)
        sy2 = jnp.where(is_deg, dy2, sy2)
        ssc = jnp.where(is_deg, ds, ssc)

        xx1 = jnp.maximum(sx1, x1c)
        yy1 = jnp.maximum(sy1, y1c)
        xx2 = jnp.minimum(sx2, x2c)
        yy2 = jnp.minimum(sy2, y2c)
        w = jnp.maximum(xx2 - xx1, f32(0.0))
        h = jnp.maximum(yy2 - yy1, f32(0.0))
        inter = w * h
        sel_area = (sx2 - sx1) * (sy2 - sy1)
        iou = inter / (areas + sel_area - inter + f32(1e-9))
        s = jnp.where((iou > f32(_IOU)) | mask2, f32(_NEG), s)

        out = (jnp.where(lane1 == 0, sx1, f32(0.0))
               + jnp.where(lane1 == 1, sy1, f32(0.0))
               + jnp.where(lane1 == 2, sx2, f32(0.0))
               + jnp.where(lane1 == 3, sy2, f32(0.0))
               + jnp.where(lane1 == 4, ssc, f32(0.0)))
        out_ref[pl.ds(i, 1), :] = out
        return (s, dx1, dy1, dx2, dy2, ds)

    zero1 = jnp.zeros((1, _LANES), f32)
    lax.fori_loop(0, _NOUT, step, (s0, zero1, zero1, zero1, zero1, zero1))


@jax.jit
def kernel(boxes, scores):
    pad = _P - _N
    comps = [
        jnp.pad(boxes[:, i], (0, pad)).reshape(_ROWS, _LANES) for i in range(4)
    ]
    s = jnp.pad(scores, (0, pad)).reshape(_ROWS, _LANES)
    out = pl.pallas_call(
        _nms_body,
        out_shape=jax.ShapeDtypeStruct((_NOUT, _LANES), jnp.float32),
    )(*comps, s)
    return out[:, :5]


# speedup vs baseline: 2.9832x; 2.9832x over previous
import functools

import jax
import jax.numpy as jnp
from jax import lax
from jax.experimental import pallas as pl
from jax.experimental.pallas import tpu as pltpu

_N = 20000
_K = 6000
_NOUT = 300
_IOU = 0.7
_SCALE = 1000.0
_ROWS = 160
_LANES = 128
_P = _ROWS * _LANES
_C = 6144
_CROWS = 48
_NEG = -1e9


def _nms_body(c0_ref, c1_ref, c2_ref, c3_ref, s_ref, out_ref,
              x1s, y1s, x2s, y2s, scs, areas_s):
    f32 = jnp.float32
    i32 = jnp.int32
    imin = jnp.int32(-2147483648)

    row_i = lax.broadcasted_iota(i32, (_ROWS, _LANES), 0)
    lane_i = lax.broadcasted_iota(i32, (_ROWS, _LANES), 1)
    flat_i = row_i * _LANES + lane_i
    flat_c = lax.broadcasted_iota(i32, (_CROWS, _LANES), 0) * _LANES \
        + lax.broadcasted_iota(i32, (_CROWS, _LANES), 1)
    lane1 = lax.broadcasted_iota(i32, (1, _LANES), 1)
    valid = flat_i < _N

    b0 = c0_ref[:] * _SCALE
    b1 = c1_ref[:] * _SCALE
    b2 = c2_ref[:] * _SCALE
    b3 = c3_ref[:] * _SCALE
    x1 = jnp.minimum(b0, b2)
    x2 = jnp.maximum(b0, b2)
    y1 = jnp.minimum(b1, b3)
    y2 = jnp.maximum(b1, b3)
    scores = s_ref[:]

    bits = lax.bitcast_convert_type(scores, i32)
    akey = bits ^ (lax.shift_right_arithmetic(bits, 31) & jnp.int32(0x7FFFFFFF))
    akey = jnp.where(valid, akey, imin)
    inv = _P - flat_i

    Tf = imin
    Ti = jnp.int32(0)
    for b in range(31, -1, -1):
        trial = (Tf ^ imin) if b == 31 else (Tf | jnp.int32(1 << b))
        cnt = jnp.sum((akey >= trial).astype(i32))
        Tf = jnp.where(cnt >= _K, trial, Tf)
    for b in range(14, -1, -1):
        trial = Ti | jnp.int32(1 << b)
        cond = (akey > Tf) | ((akey == Tf) & (inv >= trial))
        cnt = jnp.sum(cond.astype(i32))
        Ti = jnp.where(cnt >= _K, trial, Ti)
    in_set = (akey > Tf) | ((akey == Tf) & (inv >= Ti))

    bf16 = jnp.bfloat16
    mask_bf = in_set.astype(bf16)
    up = (lax.broadcasted_iota(i32, (_LANES, _LANES), 0)
          < lax.broadcasted_iota(i32, (_LANES, _LANES), 1)).astype(bf16)
    lane_excl = lax.dot_general(mask_bf, up, (((1,), (0,)), ((), ())),
                                preferred_element_type=jnp.float32)
    rowsum = jnp.sum(in_set.astype(f32), axis=1, keepdims=True)
    lo = (lax.broadcasted_iota(i32, (_ROWS, _ROWS), 0)
          > lax.broadcasted_iota(i32, (_ROWS, _ROWS), 1)).astype(bf16)
    row_excl = lax.dot_general(lo, rowsum.astype(bf16), (((1,), (0,)), ((), ())),
                               preferred_element_type=jnp.float32)
    rank = (row_excl + lane_excl).astype(i32)

    d = jnp.where(in_set, flat_i - rank, 0)

    def shift_down(v, k):
        r, l = divmod(k, _LANES)
        if l:
            a = jnp.concatenate([v[:, l:], v[:, :l]], axis=1)
            b_ = jnp.concatenate([a[1:, :], a[:1, :]], axis=0)
            v = jnp.where(lane_i < _LANES - l, a, b_)
        if r:
            v = jnp.concatenate([v[r:, :], v[:r, :]], axis=0)
        return v

    for b in range(14):
        k = 1 << b
        dr = shift_down(d, k)
        mv = (lax.shift_right_logical(dr, b) & 1) == 1
        x1 = jnp.where(mv, shift_down(x1, k), x1)
        y1 = jnp.where(mv, shift_down(y1, k), y1)
        x2 = jnp.where(mv, shift_down(x2, k), x2)
        y2 = jnp.where(mv, shift_down(y2, k), y2)
        scores = jnp.where(mv, shift_down(scores, k), scores)
        d = jnp.where(mv, dr, d)

    x1c = x1[:_CROWS]
    y1c = y1[:_CROWS]
    x2c = x2[:_CROWS]
    y2c = y2[:_CROWS]
    scc = scores[:_CROWS]
    x1s[:] = x1c
    y1s[:] = y1c
    x2s[:] = x2c
    y2s[:] = y2c
    scs[:] = scc
    areas = (x2c - x1c) * (y2c - y1c)
    areas_s[:] = areas
    s0 = jnp.where(flat_c < _K, scc, f32(_NEG))
    neg_inf = f32(-jnp.inf)

    def step(i, carry):
        s, dx1, dy1, dx2, dy2, ds = carry
        m = jnp.max(s)
        idx = jnp.min(jnp.where(s == m, flat_c, _C))
        mask2 = flat_c == idx
        row = idx // _LANES
        lmask = lane1 == (idx - row * _LANES)
        sx1 = jnp.max(jnp.where(lmask, x1s[pl.ds(row, 1), :], neg_inf))
        sy1 = jnp.max(jnp.where(lmask, y1s[pl.ds(row, 1), :], neg_inf))
        sx2 = jnp.max(jnp.where(lmask, x2s[pl.ds(row, 1), :], neg_inf))
        sy2 = jnp.max(jnp.where(lmask, y2s[pl.ds(row, 1), :], neg_inf))
        ssc = jnp.max(jnp.where(lmask, scs[pl.ds(row, 1), :], neg_inf))

        is_deg = m == f32(_NEG)
        dx1 = jnp.where(i == 0, sx1, dx1)
        dy1 = jnp.where(i == 0, sy1, dy1)
        dx2 = jnp.where(i == 0, sx2, dx2)
        dy2 = jnp.where(i == 0, sy2, dy2)
        ds = jnp.where(i == 0, ssc, ds)
        sx1 = jnp.where(is_deg, dx1, sx1)
        sy1 = jnp.where(is_deg, dy1, sy1)
        sx2 = jnp.where(is_deg, dx2, sx2)
        sy2 = jnp.where(is_deg, dy2, sy2)
        ssc = jnp.where(is_deg, ds, ssc)

        xx1 = jnp.maximum(sx1, x1s[:])
        yy1 = jnp.maximum(sy1, y1s[:])
        xx2 = jnp.minimum(sx2, x2s[:])
        yy2 = jnp.minimum(sy2, y2s[:])
        w = jnp.maximum(xx2 - xx1, f32(0.0))
        h = jnp.maximum(yy2 - yy1, f32(0.0))
        inter = w * h
        sel_area = (sx2 - sx1) * (sy2 - sy1)
        iou = inter / (areas_s[:] + sel_area - inter + f32(1e-9))
        s = jnp.where((iou > f32(_IOU)) | mask2, f32(_NEG), s)

        out = (jnp.where(lane1 == 0, sx1, f32(0.0))
               + jnp.where(lane1 == 1, sy1, f32(0.0))
               + jnp.where(lane1 == 2, sx2, f32(0.0))
               + jnp.where(lane1 == 3, sy2, f32(0.0))
               + jnp.where(lane1 == 4, ssc, f32(0.0)))
        out_ref[pl.ds(i, 1), :] = out
        return (s, dx1, dy1, dx2, dy2, ds)

    zero = f32(0.0)
    lax.fori_loop(0, _NOUT, step, (s0, zero, zero, zero, zero, zero))


@jax.jit
def kernel(boxes, scores):
    pad = _P - _N
    comps = [
        jnp.pad(boxes[:, i], (0, pad)).reshape(_ROWS, _LANES) for i in range(4)
    ]
    s = jnp.pad(scores, (0, pad)).reshape(_ROWS, _LANES)
    scratch = [pltpu.VMEM((_CROWS, _LANES), jnp.float32)] * 6
    out = pl.pallas_call(
        _nms_body,
        out_shape=jax.ShapeDtypeStruct((_NOUT, _LANES), jnp.float32),
        scratch_shapes=scratch,
    )(*comps, s)
    return out[:, :5]


# speedup vs baseline: 2.9832x; 1.0000x over previous
import functools

import jax
import jax.numpy as jnp
from jax import lax
from jax.experimental import pallas as pl
from jax.experimental.pallas import tpu as pltpu

_N = 20000
_K = 6000
_NOUT = 300
_IOU = 0.7
_SCALE = 1000.0
_ROWS = 160
_LANES = 128
_P = _ROWS * _LANES
_C = 6144
_CROWS = 48
_NEG = -1e9


def _nms_body(c0_ref, c1_ref, c2_ref, c3_ref, s_ref, out_ref,
              x1s, y1s, x2s, y2s, areas_s):
    f32 = jnp.float32
    i32 = jnp.int32
    imin = jnp.int32(-2147483648)

    row_i = lax.broadcasted_iota(i32, (_ROWS, _LANES), 0)
    lane_i = lax.broadcasted_iota(i32, (_ROWS, _LANES), 1)
    flat_i = row_i * _LANES + lane_i
    flat_c = lax.broadcasted_iota(i32, (_CROWS, _LANES), 0) * _LANES \
        + lax.broadcasted_iota(i32, (_CROWS, _LANES), 1)
    lane1 = lax.broadcasted_iota(i32, (1, _LANES), 1)
    valid = flat_i < _N

    b0 = c0_ref[:] * _SCALE
    b1 = c1_ref[:] * _SCALE
    b2 = c2_ref[:] * _SCALE
    b3 = c3_ref[:] * _SCALE
    x1 = jnp.minimum(b0, b2)
    x2 = jnp.maximum(b0, b2)
    y1 = jnp.minimum(b1, b3)
    y2 = jnp.maximum(b1, b3)
    scores = s_ref[:]

    bits = lax.bitcast_convert_type(scores, i32)
    akey = bits ^ (lax.shift_right_arithmetic(bits, 31) & jnp.int32(0x7FFFFFFF))
    akey = jnp.where(valid, akey, imin)
    inv = _P - flat_i

    Tf = imin
    Ti = jnp.int32(0)
    for b in range(31, -1, -1):
        trial = (Tf ^ imin) if b == 31 else (Tf | jnp.int32(1 << b))
        cnt = jnp.sum((akey >= trial).astype(i32))
        Tf = jnp.where(cnt >= _K, trial, Tf)
    for b in range(14, -1, -1):
        trial = Ti | jnp.int32(1 << b)
        cond = (akey > Tf) | ((akey == Tf) & (inv >= trial))
        cnt = jnp.sum(cond.astype(i32))
        Ti = jnp.where(cnt >= _K, trial, Ti)
    in_set = (akey > Tf) | ((akey == Tf) & (inv >= Ti))

    bf16 = jnp.bfloat16
    mask_bf = in_set.astype(bf16)
    up = (lax.broadcasted_iota(i32, (_LANES, _LANES), 0)
          < lax.broadcasted_iota(i32, (_LANES, _LANES), 1)).astype(bf16)
    lane_excl = lax.dot_general(mask_bf, up, (((1,), (0,)), ((), ())),
                                preferred_element_type=jnp.float32)
    rowsum = jnp.sum(in_set.astype(f32), axis=1, keepdims=True)
    lo = (lax.broadcasted_iota(i32, (_ROWS, _ROWS), 0)
          > lax.broadcasted_iota(i32, (_ROWS, _ROWS), 1)).astype(bf16)
    row_excl = lax.dot_general(lo, rowsum.astype(bf16), (((1,), (0,)), ((), ())),
                               preferred_element_type=jnp.float32)
    rank = (row_excl + lane_excl).astype(i32)

    d = jnp.where(in_set, flat_i - rank, 0)

    def shift_down(v, k):
        r, l = divmod(k, _LANES)
        if l:
            a = jnp.concatenate([v[:, l:], v[:, :l]], axis=1)
            b_ = jnp.concatenate([a[1:, :], a[:1, :]], axis=0)
            v = jnp.where(lane_i < _LANES - l, a, b_)
        if r:
            v = jnp.concatenate([v[r:, :], v[:r, :]], axis=0)
        return v

    for b in range(14):
        k = 1 << b
        dr = shift_down(d, k)
        mv = (lax.shift_right_logical(dr, b) & 1) == 1
        x1 = jnp.where(mv, shift_down(x1, k), x1)
        y1 = jnp.where(mv, shift_down(y1, k), y1)
        x2 = jnp.where(mv, shift_down(x2, k), x2)
        y2 = jnp.where(mv, shift_down(y2, k), y2)
        scores = jnp.where(mv, shift_down(scores, k), scores)
        d = jnp.where(mv, dr, d)

    x1c = x1[:_CROWS]
    y1c = y1[:_CROWS]
    x2c = x2[:_CROWS]
    y2c = y2[:_CROWS]
    scc = scores[:_CROWS]
    x1s[:] = x1c
    y1s[:] = y1c
    x2s[:] = x2c
    y2s[:] = y2c
    areas = (x2c - x1c) * (y2c - y1c)
    areas_s[:] = areas
    s0 = jnp.where(flat_c < _K, scc, f32(_NEG))
    neg_inf = f32(-jnp.inf)

    def step(i, carry):
        s, dx1, dy1, dx2, dy2, ds = carry
        m = jnp.max(s)
        idx = jnp.min(jnp.where(s == m, flat_c, _C))
        mask2 = flat_c == idx
        row = idx // _LANES
        lmask = lane1 == (idx - row * _LANES)
        sx1 = jnp.max(jnp.where(lmask, x1s[pl.ds(row, 1), :], neg_inf))
        sy1 = jnp.max(jnp.where(lmask, y1s[pl.ds(row, 1), :], neg_inf))
        sx2 = jnp.max(jnp.where(lmask, x2s[pl.ds(row, 1), :], neg_inf))
        sy2 = jnp.max(jnp.where(lmask, y2s[pl.ds(row, 1), :], neg_inf))
        ssc = m

        is_deg = m == f32(_NEG)
        dx1 = jnp.where(i == 0, sx1, dx1)
        dy1 = jnp.where(i == 0, sy1, dy1)
        dx2 = jnp.where(i == 0, sx2, dx2)
        dy2 = jnp.where(i == 0, sy2, dy2)
        ds = jnp.where(i == 0, ssc, ds)
        sx1 = jnp.where(is_deg, dx1, sx1)
        sy1 = jnp.where(is_deg, dy1, sy1)
        sx2 = jnp.where(is_deg, dx2, sx2)
        sy2 = jnp.where(is_deg, dy2, sy2)
        ssc = jnp.where(is_deg, ds, ssc)

        xx1 = jnp.maximum(sx1, x1s[:])
        yy1 = jnp.maximum(sy1, y1s[:])
        xx2 = jnp.minimum(sx2, x2s[:])
        yy2 = jnp.minimum(sy2, y2s[:])
        w = jnp.maximum(xx2 - xx1, f32(0.0))
        h = jnp.maximum(yy2 - yy1, f32(0.0))
        inter = w * h
        sel_area = (sx2 - sx1) * (sy2 - sy1)
        iou = inter / (areas_s[:] + sel_area - inter + f32(1e-9))
        s = jnp.where((iou > f32(_IOU)) | mask2, f32(_NEG), s)

        out = (jnp.where(lane1 == 0, sx1, f32(0.0))
               + jnp.where(lane1 == 1, sy1, f32(0.0))
               + jnp.where(lane1 == 2, sx2, f32(0.0))
               + jnp.where(lane1 == 3, sy2, f32(0.0))
               + jnp.where(lane1 == 4, ssc, f32(0.0)))
        out_ref[pl.ds(i, 1), :] = out
        return (s, dx1, dy1, dx2, dy2, ds)

    zero = f32(0.0)
    lax.fori_loop(0, _NOUT, step, (s0, zero, zero, zero, zero, zero))


@jax.jit
def kernel(boxes, scores):
    pad = _P - _N
    comps = [
        jnp.pad(boxes[:, i], (0, pad)).reshape(_ROWS, _LANES) for i in range(4)
    ]
    s = jnp.pad(scores, (0, pad)).reshape(_ROWS, _LANES)
    scratch = [pltpu.VMEM((_CROWS, _LANES), jnp.float32)] * 5
    out = pl.pallas_call(
        _nms_body,
        out_shape=jax.ShapeDtypeStruct((_NOUT, _LANES), jnp.float32),
        scratch_shapes=scratch,
    )(*comps, s)
    return out[:, :5]
